# speedup vs baseline: 8.0248x; 8.0248x over previous
import functools

import jax
import jax.numpy as jnp
from jax import lax
from jax.experimental import pallas as pl
from jax.experimental.pallas import tpu as pltpu
from jax.experimental.pallas import tpu_sc as plsc

N_NODES = 10000
N_EDGES = 320000
D = 128
NC, NS = 2, 16
NW = NC * NS
NPAD = 10240
CHUNK = 128
EPW = 10240
NCHUNK = EPW // CHUNK
EPAD = NW * EPW
ROWS_PER_TILE = NPAD // NS
DEGW = 16
R = 1024


def _mesh():
    return plsc.VectorSubcoreMesh(core_axis_name="c", subcore_axis_name="s",
                                  num_cores=NC, num_subcores=NS)


def _sc_degree(dst_r):
    @functools.partial(
        pl.kernel,
        out_type=jax.ShapeDtypeStruct((NC, NPAD, DEGW), jnp.float32),
        mesh=_mesh(),
        scratch_types=[
            pltpu.VMEM((NCHUNK, CHUNK), jnp.int32),
            pltpu.VMEM((CHUNK, DEGW), jnp.float32),
            pltpu.VMEM((CHUNK, DEGW), jnp.float32),
            pltpu.VMEM_SHARED((NPAD, DEGW), jnp.float32),
        ],
    )
    def deg_kernel(dst_hbm, out_hbm, dstv, ones_v, zero_v, acc):
        c = lax.axis_index("c")
        s = lax.axis_index("s")
        wid = c * NS + s

        @pl.loop(0, CHUNK)
        def _fill(i):
            ones_v[i, :] = jnp.ones((DEGW,), jnp.float32)
            zero_v[i, :] = jnp.zeros((DEGW,), jnp.float32)

        @pl.loop(0, ROWS_PER_TILE // CHUNK)
        def _zero(i):
            pltpu.sync_copy(zero_v,
                            acc.at[pl.ds(s * ROWS_PER_TILE + i * CHUNK, CHUNK)])

        plsc.subcore_barrier()
        pltpu.sync_copy(dst_hbm.at[wid], dstv)

        @pl.loop(0, NCHUNK)
        def _scatter(j):
            pltpu.sync_copy(ones_v, acc.at[dstv.at[j]], add=True)

        plsc.subcore_barrier()
        pltpu.sync_copy(acc.at[pl.ds(s * ROWS_PER_TILE, ROWS_PER_TILE)],
                        out_hbm.at[c, pl.ds(s * ROWS_PER_TILE, ROWS_PER_TILE)])

    return deg_kernel(dst_r)


def _sc_hop(ytil, src_r, dst_r):
    @functools.partial(
        pl.kernel,
        out_type=jax.ShapeDtypeStruct((NC, NPAD, D), jnp.float32),
        mesh=_mesh(),
        scratch_types=[
            pltpu.VMEM((NCHUNK, CHUNK), jnp.int32),
            pltpu.VMEM((NCHUNK, CHUNK), jnp.int32),
            pltpu.VMEM((CHUNK, D), jnp.float32),
            pltpu.VMEM_SHARED((NPAD, D), jnp.float32),
            pltpu.SemaphoreType.DMA,
        ],
    )
    def hop_kernel(ytil_hbm, src_hbm, dst_hbm, out_hbm, srcv, dstv, rows, acc, sem):
        c = lax.axis_index("c")
        s = lax.axis_index("s")
        wid = c * NS + s

        @pl.loop(0, CHUNK)
        def _zfill(i):
            for k in range(D // 16):
                rows[i, pl.ds(k * 16, 16)] = jnp.zeros((16,), jnp.float32)

        @pl.loop(0, ROWS_PER_TILE // CHUNK)
        def _zero(i):
            pltpu.sync_copy(rows,
                            acc.at[pl.ds(s * ROWS_PER_TILE + i * CHUNK, CHUNK)])

        plsc.subcore_barrier()
        pltpu.sync_copy(src_hbm.at[wid], srcv)
        pltpu.sync_copy(dst_hbm.at[wid], dstv)

        @pl.loop(0, NCHUNK)
        def _edges(j):
            pltpu.async_copy(ytil_hbm.at[srcv.at[j]], rows, sem).wait()
            pltpu.sync_copy(rows, acc.at[dstv.at[j]], add=True)

        plsc.subcore_barrier()
        pltpu.sync_copy(acc.at[pl.ds(s * ROWS_PER_TILE, ROWS_PER_TILE)],
                        out_hbm.at[c, pl.ds(s * ROWS_PER_TILE, ROWS_PER_TILE)])

    return hop_kernel(ytil, src_r, dst_r)


def _dinv_block(deg_ref, i):
    degs = deg_ref[0, :, 0:1] + deg_ref[1, :, 0:1] + 1.0
    rows = i * R + lax.broadcasted_iota(jnp.int32, (R, 1), 0)
    return jnp.where(rows < N_NODES, lax.rsqrt(degs), 0.0)


def _row_spec():
    return pl.BlockSpec((R, D), lambda i: (i, 0))


def _mat_spec():
    return pl.BlockSpec((D, D), lambda i: (0, 0))


def _bias_spec():
    return pl.BlockSpec((1, D), lambda i: (0, 0))


def _deg_spec():
    return pl.BlockSpec((NC, R, DEGW), lambda i: (0, i, 0))


def _acc_spec():
    return pl.BlockSpec((NC, R, D), lambda i: (0, i, 0))


def _tc_pre(xpad, W0, b0, W1, b1, deg):
    def body(x_ref, w0_ref, b0_ref, w1_ref, b1_ref, deg_ref, h0_ref, yt_ref):
        i = pl.program_id(0)
        dinv = _dinv_block(deg_ref, i)
        x = x_ref[...]
        h0 = jnp.maximum(
            jnp.dot(x, w0_ref[...], precision=lax.Precision.HIGHEST) + b0_ref[...],
            0.0)
        y1 = jnp.dot(h0, w1_ref[...], precision=lax.Precision.HIGHEST) + b1_ref[...]
        h0_ref[...] = h0
        yt_ref[...] = dinv * y1

    return pl.pallas_call(
        body,
        grid=(NPAD // R,),
        in_specs=[_row_spec(), _mat_spec(), _bias_spec(), _mat_spec(),
                  _bias_spec(), _deg_spec()],
        out_specs=[_row_spec(), _row_spec()],
        out_shape=[jax.ShapeDtypeStruct((NPAD, D), jnp.float32)] * 2,
    )(xpad, W0, b0.reshape(1, D), W1, b1.reshape(1, D), deg)


def _tc_mid(acc, yt, s_prev, deg, W, b):
    def body(acc_ref, yt_ref, s_ref, deg_ref, w_ref, b_ref, sout_ref, ytout_ref):
        i = pl.program_id(0)
        dinv = _dinv_block(deg_ref, i)
        a = acc_ref[0] + acc_ref[1] + yt_ref[...]
        x = jnp.maximum(dinv * a, 0.0)
        sout_ref[...] = s_ref[...] + x
        y = jnp.dot(x, w_ref[...], precision=lax.Precision.HIGHEST) + b_ref[...]
        ytout_ref[...] = dinv * y

    return pl.pallas_call(
        body,
        grid=(NPAD // R,),
        in_specs=[_acc_spec(), _row_spec(), _row_spec(), _deg_spec(),
                  _mat_spec(), _bias_spec()],
        out_specs=[_row_spec(), _row_spec()],
        out_shape=[jax.ShapeDtypeStruct((NPAD, D), jnp.float32)] * 2,
    )(acc, yt, s_prev, deg, W, b.reshape(1, D))


def _tc_final(acc, yt, s_prev, deg):
    def body(acc_ref, yt_ref, s_ref, deg_ref, out_ref):
        i = pl.program_id(0)
        dinv = _dinv_block(deg_ref, i)
        a = acc_ref[0] + acc_ref[1] + yt_ref[...]
        out_ref[...] = s_ref[...] + jnp.maximum(dinv * a, 0.0)

    return pl.pallas_call(
        body,
        grid=(NPAD // R,),
        in_specs=[_acc_spec(), _row_spec(), _row_spec(), _deg_spec()],
        out_specs=_row_spec(),
        out_shape=jax.ShapeDtypeStruct((NPAD, D), jnp.float32),
    )(acc, yt, s_prev, deg)


def kernel(features, edge_index, W0, b0, W1, b1, W2, b2, W3, b3):
    src = edge_index[0]
    dst = edge_index[1]
    fill = jnp.full((EPAD - N_EDGES,), N_NODES, dtype=src.dtype)
    src_r = jnp.concatenate([src, fill]).reshape(NW, NCHUNK, CHUNK)
    dst_r = jnp.concatenate([dst, fill]).reshape(NW, NCHUNK, CHUNK)
    xpad = jnp.zeros((NPAD, D), features.dtype).at[:N_NODES].set(features)

    deg = _sc_degree(dst_r)
    s_run, yt = _tc_pre(xpad, W0, b0, W1, b1, deg)
    for (W, b) in ((W2, b2), (W3, b3)):
        acc = _sc_hop(yt, src_r, dst_r)
        s_run, yt = _tc_mid(acc, yt, s_run, deg, W, b)
    acc = _sc_hop(yt, src_r, dst_r)
    out = _tc_final(acc, yt, s_run, deg)
    return out[:N_NODES]


# speedup vs baseline: 8.8324x; 1.1006x over previous
import functools

import jax
import jax.numpy as jnp
from jax import lax
from jax.experimental import pallas as pl
from jax.experimental.pallas import tpu as pltpu
from jax.experimental.pallas import tpu_sc as plsc

N_NODES = 10000
N_EDGES = 320000
D = 128
NC, NS = 2, 16
NW = NC * NS
NPAD = 10240
CHUNK = 128
EPAD = 327680
NCHUNK = EPAD // NW // CHUNK
ROWS_PER_TILE = NPAD // NS
DEGW = 16
WIN = 8
NWINDOW = NCHUNK // WIN
R = 1024


def _mesh():
    return plsc.VectorSubcoreMesh(core_axis_name="c", subcore_axis_name="s",
                                  num_cores=NC, num_subcores=NS)


def _sc_degree(dst_r):
    @functools.partial(
        pl.kernel,
        out_type=jax.ShapeDtypeStruct((NC, NPAD, DEGW), jnp.float32),
        mesh=_mesh(),
        scratch_types=[
            pltpu.VMEM((NCHUNK, CHUNK), jnp.int32),
            pltpu.VMEM((CHUNK, DEGW), jnp.float32),
            pltpu.VMEM((CHUNK, DEGW), jnp.float32),
            pltpu.VMEM_SHARED((NPAD, DEGW), jnp.float32),
        ],
    )
    def deg_kernel(dst_hbm, out_hbm, dstv, ones_v, zero_v, acc):
        c = lax.axis_index("c")
        s = lax.axis_index("s")
        wid = c * NS + s

        @pl.loop(0, CHUNK)
        def _fill(i):
            ones_v[i, :] = jnp.ones((DEGW,), jnp.float32)
            zero_v[i, :] = jnp.zeros((DEGW,), jnp.float32)

        @pl.loop(0, ROWS_PER_TILE // CHUNK)
        def _zero(i):
            pltpu.sync_copy(zero_v,
                            acc.at[pl.ds(s * ROWS_PER_TILE + i * CHUNK, CHUNK)])

        plsc.subcore_barrier()
        pltpu.sync_copy(dst_hbm.at[wid], dstv)

        @pl.loop(0, NCHUNK)
        def _scatter(j):
            pltpu.sync_copy(ones_v, acc.at[dstv.at[j]], add=True)

        plsc.subcore_barrier()
        pltpu.sync_copy(acc.at[pl.ds(s * ROWS_PER_TILE, ROWS_PER_TILE)],
                        out_hbm.at[c, pl.ds(s * ROWS_PER_TILE, ROWS_PER_TILE)])

    return deg_kernel(dst_r)


def _sc_hop(ytil, src_r, dst_r):
    @functools.partial(
        pl.kernel,
        out_type=jax.ShapeDtypeStruct((NC, NPAD, D), jnp.float32),
        mesh=_mesh(),
        scratch_types=[
            pltpu.VMEM((3, WIN, CHUNK), jnp.int32),
            pltpu.VMEM((NCHUNK, CHUNK), jnp.int32),
            pltpu.VMEM((2, CHUNK, D), jnp.float32),
            pltpu.VMEM_SHARED((NPAD, D), jnp.float32),
            pltpu.SemaphoreType.DMA,
            pltpu.SemaphoreType.DMA,
            pltpu.SemaphoreType.DMA,
            pltpu.SemaphoreType.DMA,
            pltpu.SemaphoreType.DMA,
            pltpu.SemaphoreType.DMA,
            pltpu.SemaphoreType.DMA,
        ],
    )
    def hop_kernel(ytil_hbm, src_hbm, dst_hbm, out_hbm, srcv, dstv, rows, acc,
                   gsem0, gsem1, ssem0, ssem1, isem0, isem1, isem2):
        gsems = (gsem0, gsem1)
        ssems = (ssem0, ssem1)
        isems = (isem0, isem1, isem2)
        c = lax.axis_index("c")
        s = lax.axis_index("s")
        wid = c * NS + s

        @pl.loop(0, CHUNK)
        def _zfill(i):
            for k in range(D // 16):
                rows[0, i, pl.ds(k * 16, 16)] = jnp.zeros((16,), jnp.float32)

        @pl.loop(0, ROWS_PER_TILE // CHUNK)
        def _zero(i):
            pltpu.sync_copy(rows.at[0],
                            acc.at[pl.ds(s * ROWS_PER_TILE + i * CHUNK, CHUNK)])

        plsc.subcore_barrier()
        pltpu.sync_copy(dst_hbm.at[wid], dstv)
        pltpu.sync_copy(src_hbm.at[wid, pl.ds(0, WIN)], srcv.at[0])
        pltpu.async_copy(src_hbm.at[wid, pl.ds(WIN, WIN)], srcv.at[1], isems[1])
        pltpu.async_copy(ytil_hbm.at[srcv.at[0, 0]], rows.at[0], gsems[0])

        for w in range(NWINDOW):
            bank = w % 3
            if w + 2 < NWINDOW:
                pltpu.async_copy(src_hbm.at[wid, pl.ds((w + 2) * WIN, WIN)],
                                 srcv.at[(w + 2) % 3], isems[(w + 2) % 3])
            for t in range(WIN):
                b = t % 2
                j = w * WIN + t
                pltpu.make_async_copy(ytil_hbm.at[srcv.at[bank, t]],
                                      rows.at[b], gsems[b]).wait()
                pltpu.async_copy(rows.at[b], acc.at[dstv.at[j]], ssems[b],
                                 add=True)
                if j >= 1:
                    pltpu.make_async_copy(rows.at[1 - b],
                                          acc.at[dstv.at[j - 1]],
                                          ssems[1 - b]).wait()
                if t < WIN - 1:
                    pltpu.async_copy(ytil_hbm.at[srcv.at[bank, t + 1]],
                                     rows.at[1 - b], gsems[1 - b])
                elif w + 1 < NWINDOW:
                    nb = (w + 1) % 3
                    pltpu.make_async_copy(
                        src_hbm.at[wid, pl.ds((w + 1) * WIN, WIN)],
                        srcv.at[nb], isems[nb]).wait()
                    pltpu.async_copy(ytil_hbm.at[srcv.at[nb, 0]],
                                     rows.at[1 - b], gsems[1 - b])

        pltpu.make_async_copy(rows.at[(NCHUNK - 1) % 2],
                              acc.at[dstv.at[NCHUNK - 1]],
                              ssems[(NCHUNK - 1) % 2]).wait()
        plsc.subcore_barrier()
        pltpu.sync_copy(acc.at[pl.ds(s * ROWS_PER_TILE, ROWS_PER_TILE)],
                        out_hbm.at[c, pl.ds(s * ROWS_PER_TILE, ROWS_PER_TILE)])

    return hop_kernel(ytil, src_r, dst_r)


def _dinv_block(deg_ref, i):
    degs = deg_ref[0, :, 0:1] + deg_ref[1, :, 0:1] + 1.0
    rows = i * R + lax.broadcasted_iota(jnp.int32, (R, 1), 0)
    return jnp.where(rows < N_NODES, lax.rsqrt(degs), 0.0)


def _row_spec():
    return pl.BlockSpec((R, D), lambda i: (i, 0))


def _mat_spec():
    return pl.BlockSpec((D, D), lambda i: (0, 0))


def _bias_spec():
    return pl.BlockSpec((1, D), lambda i: (0, 0))


def _deg_spec():
    return pl.BlockSpec((NC, R, DEGW), lambda i: (0, i, 0))


def _acc_spec():
    return pl.BlockSpec((NC, R, D), lambda i: (0, i, 0))


def _tc_pre(xpad, W0, b0, W1, b1, deg):
    def body(x_ref, w0_ref, b0_ref, w1_ref, b1_ref, deg_ref, h0_ref, yt_ref):
        i = pl.program_id(0)
        dinv = _dinv_block(deg_ref, i)
        x = x_ref[...]
        h0 = jnp.maximum(
            jnp.dot(x, w0_ref[...], precision=lax.Precision.HIGHEST) + b0_ref[...],
            0.0)
        y1 = jnp.dot(h0, w1_ref[...], precision=lax.Precision.HIGHEST) + b1_ref[...]
        h0_ref[...] = h0
        yt_ref[...] = dinv * y1

    return pl.pallas_call(
        body,
        grid=(NPAD // R,),
        in_specs=[_row_spec(), _mat_spec(), _bias_spec(), _mat_spec(),
                  _bias_spec(), _deg_spec()],
        out_specs=[_row_spec(), _row_spec()],
        out_shape=[jax.ShapeDtypeStruct((NPAD, D), jnp.float32)] * 2,
    )(xpad, W0, b0.reshape(1, D), W1, b1.reshape(1, D), deg)


def _tc_mid(acc, yt, s_prev, deg, W, b):
    def body(acc_ref, yt_ref, s_ref, deg_ref, w_ref, b_ref, sout_ref, ytout_ref):
        i = pl.program_id(0)
        dinv = _dinv_block(deg_ref, i)
        a = acc_ref[0] + acc_ref[1] + yt_ref[...]
        x = jnp.maximum(dinv * a, 0.0)
        sout_ref[...] = s_ref[...] + x
        y = jnp.dot(x, w_ref[...], precision=lax.Precision.HIGHEST) + b_ref[...]
        ytout_ref[...] = dinv * y

    return pl.pallas_call(
        body,
        grid=(NPAD // R,),
        in_specs=[_acc_spec(), _row_spec(), _row_spec(), _deg_spec(),
                  _mat_spec(), _bias_spec()],
        out_specs=[_row_spec(), _row_spec()],
        out_shape=[jax.ShapeDtypeStruct((NPAD, D), jnp.float32)] * 2,
    )(acc, yt, s_prev, deg, W, b.reshape(1, D))


def _tc_final(acc, yt, s_prev, deg):
    def body(acc_ref, yt_ref, s_ref, deg_ref, out_ref):
        i = pl.program_id(0)
        dinv = _dinv_block(deg_ref, i)
        a = acc_ref[0] + acc_ref[1] + yt_ref[...]
        out_ref[...] = s_ref[...] + jnp.maximum(dinv * a, 0.0)

    return pl.pallas_call(
        body,
        grid=(NPAD // R,),
        in_specs=[_acc_spec(), _row_spec(), _row_spec(), _deg_spec()],
        out_specs=_row_spec(),
        out_shape=jax.ShapeDtypeStruct((NPAD, D), jnp.float32),
    )(acc, yt, s_prev, deg)


def kernel(features, edge_index, W0, b0, W1, b1, W2, b2, W3, b3):
    src = edge_index[0]
    dst = edge_index[1]
    fill = jnp.full((EPAD - N_EDGES,), N_NODES, dtype=src.dtype)
    src_r = jnp.concatenate([src, fill]).reshape(NW, NCHUNK, CHUNK)
    dst_r = jnp.concatenate([dst, fill]).reshape(NW, NCHUNK, CHUNK)
    xpad = jnp.zeros((NPAD, D), features.dtype).at[:N_NODES].set(features)

    deg = _sc_degree(dst_r)
    s_run, yt = _tc_pre(xpad, W0, b0, W1, b1, deg)
    for (W, b) in ((W2, b2), (W3, b3)):
        acc = _sc_hop(yt, src_r, dst_r)
        s_run, yt = _tc_mid(acc, yt, s_run, deg, W, b)
    acc = _sc_hop(yt, src_r, dst_r)
    out = _tc_final(acc, yt, s_run, deg)
    return out[:N_NODES]


# speedup vs baseline: 22.9587x; 2.5994x over previous
import functools

import jax
import jax.numpy as jnp
from jax import lax
from jax.experimental import pallas as pl
from jax.experimental.pallas import tpu as pltpu
from jax.experimental.pallas import tpu_sc as plsc

N_NODES = 10000
N_EDGES = 320000
D = 128
NC, NS = 2, 16
NW = NC * NS
NPAD = 10240
CHUNK = 128
EPAD = 327680
NCHUNK = EPAD // NW // CHUNK
ROWS_PER_TILE = NPAD // NS
DEGW = 16
WIN = 8
NWINDOW = NCHUNK // WIN
R = 1024


def _mesh():
    return plsc.VectorSubcoreMesh(core_axis_name="c", subcore_axis_name="s",
                                  num_cores=NC, num_subcores=NS)


def _sc_degree(dst_r):
    @functools.partial(
        pl.kernel,
        out_type=jax.ShapeDtypeStruct((NC, NPAD, DEGW), jnp.float32),
        mesh=_mesh(),
        scratch_types=[
            pltpu.VMEM((NCHUNK, CHUNK), jnp.int32),
            pltpu.VMEM((CHUNK, DEGW), jnp.float32),
            pltpu.VMEM((CHUNK, DEGW), jnp.float32),
            pltpu.VMEM_SHARED((NPAD, DEGW), jnp.float32),
        ],
    )
    def deg_kernel(dst_hbm, out_hbm, dstv, ones_v, zero_v, acc):
        c = lax.axis_index("c")
        s = lax.axis_index("s")
        wid = c * NS + s

        @pl.loop(0, CHUNK)
        def _fill(i):
            ones_v[i, :] = jnp.ones((DEGW,), jnp.float32)
            zero_v[i, :] = jnp.zeros((DEGW,), jnp.float32)

        @pl.loop(0, ROWS_PER_TILE // CHUNK)
        def _zero(i):
            pltpu.sync_copy(zero_v,
                            acc.at[pl.ds(s * ROWS_PER_TILE + i * CHUNK, CHUNK)])

        plsc.subcore_barrier()
        pltpu.sync_copy(dst_hbm.at[wid], dstv)

        @pl.loop(0, NCHUNK)
        def _scatter(j):
            pltpu.sync_copy(ones_v, acc.at[dstv.at[j]], add=True)

        plsc.subcore_barrier()
        pltpu.sync_copy(acc.at[pl.ds(s * ROWS_PER_TILE, ROWS_PER_TILE)],
                        out_hbm.at[c, pl.ds(s * ROWS_PER_TILE, ROWS_PER_TILE)])

    return deg_kernel(dst_r)


def _sc_hop(ytil, src_r, dst_r):
    @functools.partial(
        pl.kernel,
        out_type=jax.ShapeDtypeStruct((NC, NPAD, D), jnp.float32),
        mesh=_mesh(),
        scratch_types=[
            pltpu.VMEM((3, WIN, CHUNK), jnp.int32),
            pltpu.VMEM((NCHUNK, CHUNK), jnp.int32),
            pltpu.VMEM((2, CHUNK, D), jnp.float32),
            pltpu.VMEM_SHARED((NPAD, D), jnp.float32),
            pltpu.SemaphoreType.DMA,
            pltpu.SemaphoreType.DMA,
            pltpu.SemaphoreType.DMA,
            pltpu.SemaphoreType.DMA,
            pltpu.SemaphoreType.DMA,
            pltpu.SemaphoreType.DMA,
            pltpu.SemaphoreType.DMA,
        ],
    )
    def hop_kernel(ytil_hbm, src_hbm, dst_hbm, out_hbm, srcv, dstv, rows, acc,
                   gsem0, gsem1, ssem0, ssem1, isem0, isem1, isem2):
        gsems = (gsem0, gsem1)
        ssems = (ssem0, ssem1)
        isems = (isem0, isem1, isem2)
        c = lax.axis_index("c")
        s = lax.axis_index("s")
        wid = c * NS + s

        @pl.loop(0, CHUNK)
        def _zfill(i):
            for k in range(D // 16):
                rows[0, i, pl.ds(k * 16, 16)] = jnp.zeros((16,), jnp.float32)

        @pl.loop(0, ROWS_PER_TILE // CHUNK)
        def _zero(i):
            pltpu.sync_copy(rows.at[0],
                            acc.at[pl.ds(s * ROWS_PER_TILE + i * CHUNK, CHUNK)])

        plsc.subcore_barrier()
        pltpu.sync_copy(dst_hbm.at[wid], dstv)
        pltpu.sync_copy(src_hbm.at[wid, pl.ds(0, WIN)], srcv.at[0])
        pltpu.async_copy(src_hbm.at[wid, pl.ds(WIN, WIN)], srcv.at[1], isems[1])
        pltpu.async_copy(ytil_hbm.at[srcv.at[0, 0]], rows.at[0], gsems[0])

        for w in range(NWINDOW):
            bank = w % 3
            if w + 2 < NWINDOW:
                pltpu.async_copy(src_hbm.at[wid, pl.ds((w + 2) * WIN, WIN)],
                                 srcv.at[(w + 2) % 3], isems[(w + 2) % 3])
            for t in range(WIN):
                b = t % 2
                j = w * WIN + t
                pltpu.make_async_copy(ytil_hbm.at[srcv.at[bank, t]],
                                      rows.at[b], gsems[b]).wait()
                pltpu.async_copy(rows.at[b], acc.at[dstv.at[j]], ssems[b],
                                 add=True)
                if j >= 1:
                    pltpu.make_async_copy(rows.at[1 - b],
                                          acc.at[dstv.at[j - 1]],
                                          ssems[1 - b]).wait()
                if t < WIN - 1:
                    pltpu.async_copy(ytil_hbm.at[srcv.at[bank, t + 1]],
                                     rows.at[1 - b], gsems[1 - b])
                elif w + 1 < NWINDOW:
                    nb = (w + 1) % 3
                    pltpu.make_async_copy(
                        src_hbm.at[wid, pl.ds((w + 1) * WIN, WIN)],
                        srcv.at[nb], isems[nb]).wait()
                    pltpu.async_copy(ytil_hbm.at[srcv.at[nb, 0]],
                                     rows.at[1 - b], gsems[1 - b])

        pltpu.make_async_copy(rows.at[(NCHUNK - 1) % 2],
                              acc.at[dstv.at[NCHUNK - 1]],
                              ssems[(NCHUNK - 1) % 2]).wait()
        plsc.subcore_barrier()
        pltpu.sync_copy(acc.at[pl.ds(s * ROWS_PER_TILE, ROWS_PER_TILE)],
                        out_hbm.at[c, pl.ds(s * ROWS_PER_TILE, ROWS_PER_TILE)])

    return hop_kernel(ytil, src_r, dst_r)


def _dinv_block(deg_ref, i):
    degs = deg_ref[0, :, 0:1] + deg_ref[1, :, 0:1] + 1.0
    rows = i * R + lax.broadcasted_iota(jnp.int32, (R, 1), 0)
    return jnp.where(rows < N_NODES, lax.rsqrt(degs), 0.0)


def _row_spec():
    return pl.BlockSpec((R, D), lambda i: (i, 0))


def _mat_spec():
    return pl.BlockSpec((D, D), lambda i: (0, 0))


def _bias_spec():
    return pl.BlockSpec((1, D), lambda i: (0, 0))


def _deg_spec():
    return pl.BlockSpec((NC, R, DEGW), lambda i: (0, i, 0))


def _acc_spec():
    return pl.BlockSpec((NC, R, D), lambda i: (0, i, 0))


def _tc_pre(xpad, W0, b0, W1, b1, deg):
    def body(x_ref, w0_ref, b0_ref, w1_ref, b1_ref, deg_ref, h0_ref, yt_ref):
        i = pl.program_id(0)
        dinv = _dinv_block(deg_ref, i)
        x = x_ref[...]
        h0 = jnp.maximum(
            jnp.dot(x, w0_ref[...], precision=lax.Precision.HIGHEST) + b0_ref[...],
            0.0)
        y1 = jnp.dot(h0, w1_ref[...], precision=lax.Precision.HIGHEST) + b1_ref[...]
        h0_ref[...] = h0
        yt_ref[...] = dinv * y1

    return pl.pallas_call(
        body,
        grid=(NPAD // R,),
        in_specs=[_row_spec(), _mat_spec(), _bias_spec(), _mat_spec(),
                  _bias_spec(), _deg_spec()],
        out_specs=[_row_spec(), _row_spec()],
        out_shape=[jax.ShapeDtypeStruct((NPAD, D), jnp.float32)] * 2,
    )(xpad, W0, b0.reshape(1, D), W1, b1.reshape(1, D), deg)


def _tc_mid(acc, yt, s_prev, deg, W, b):
    def body(acc_ref, yt_ref, s_ref, deg_ref, w_ref, b_ref, sout_ref, ytout_ref):
        i = pl.program_id(0)
        dinv = _dinv_block(deg_ref, i)
        a = acc_ref[0] + acc_ref[1] + yt_ref[...]
        x = jnp.maximum(dinv * a, 0.0)
        sout_ref[...] = s_ref[...] + x
        y = jnp.dot(x, w_ref[...], precision=lax.Precision.HIGHEST) + b_ref[...]
        ytout_ref[...] = dinv * y

    return pl.pallas_call(
        body,
        grid=(NPAD // R,),
        in_specs=[_acc_spec(), _row_spec(), _row_spec(), _deg_spec(),
                  _mat_spec(), _bias_spec()],
        out_specs=[_row_spec(), _row_spec()],
        out_shape=[jax.ShapeDtypeStruct((NPAD, D), jnp.float32)] * 2,
    )(acc, yt, s_prev, deg, W, b.reshape(1, D))


def _tc_final(acc, yt, s_prev, deg):
    def body(acc_ref, yt_ref, s_ref, deg_ref, out_ref):
        i = pl.program_id(0)
        dinv = _dinv_block(deg_ref, i)
        a = acc_ref[0] + acc_ref[1] + yt_ref[...]
        out_ref[...] = s_ref[...] + jnp.maximum(dinv * a, 0.0)

    return pl.pallas_call(
        body,
        grid=(NPAD // R,),
        in_specs=[_acc_spec(), _row_spec(), _row_spec(), _deg_spec()],
        out_specs=_row_spec(),
        out_shape=jax.ShapeDtypeStruct((NPAD, D), jnp.float32),
    )(acc, yt, s_prev, deg)


def kernel(features, edge_index, W0, b0, W1, b1, W2, b2, W3, b3):
    src = edge_index[0]
    dst = edge_index[1]
    fill = (N_NODES +
            jnp.arange(EPAD - N_EDGES, dtype=src.dtype) % (NPAD - N_NODES))
    src_r = jnp.concatenate([src, fill]).reshape(NW, NCHUNK, CHUNK)
    dst_r = jnp.concatenate([dst, fill]).reshape(NW, NCHUNK, CHUNK)
    xpad = jnp.zeros((NPAD, D), features.dtype).at[:N_NODES].set(features)

    deg = _sc_degree(dst_r)
    s_run, yt = _tc_pre(xpad, W0, b0, W1, b1, deg)
    for (W, b) in ((W2, b2), (W3, b3)):
        acc = _sc_hop(yt, src_r, dst_r)
        s_run, yt = _tc_mid(acc, yt, s_run, deg, W, b)
    acc = _sc_hop(yt, src_r, dst_r)
    out = _tc_final(acc, yt, s_run, deg)
    return out[:N_NODES]


# speedup vs baseline: 23.4140x; 1.0198x over previous
import functools

import jax
import jax.numpy as jnp
from jax import lax
from jax.experimental import pallas as pl
from jax.experimental.pallas import tpu as pltpu
from jax.experimental.pallas import tpu_sc as plsc

N_NODES = 10000
N_EDGES = 320000
D = 128
NC, NS = 2, 16
NW = NC * NS
NPAD = 10240
CHUNK = 128
EPAD = 327680
NCHUNK = EPAD // NW // CHUNK
HCH = CHUNK // 2
ROWS_PER_TILE = NPAD // NS
DEGW = 16
WIN = 8
NWINDOW = NCHUNK // WIN
R = 1024


def _mesh():
    return plsc.VectorSubcoreMesh(core_axis_name="c", subcore_axis_name="s",
                                  num_cores=NC, num_subcores=NS)


def _sc_degree(dst_r):
    @functools.partial(
        pl.kernel,
        out_type=jax.ShapeDtypeStruct((NC, NPAD, DEGW), jnp.float32),
        mesh=_mesh(),
        scratch_types=[
            pltpu.VMEM((NCHUNK, CHUNK), jnp.int32),
            pltpu.VMEM((CHUNK, DEGW), jnp.float32),
            pltpu.VMEM((CHUNK, DEGW), jnp.float32),
            pltpu.VMEM_SHARED((NPAD, DEGW), jnp.float32),
        ],
    )
    def deg_kernel(dst_hbm, out_hbm, dstv, ones_v, zero_v, acc):
        c = lax.axis_index("c")
        s = lax.axis_index("s")
        wid = c * NS + s

        @pl.loop(0, CHUNK)
        def _fill(i):
            ones_v[i, :] = jnp.ones((DEGW,), jnp.float32)
            zero_v[i, :] = jnp.zeros((DEGW,), jnp.float32)

        @pl.loop(0, ROWS_PER_TILE // CHUNK)
        def _zero(i):
            pltpu.sync_copy(zero_v,
                            acc.at[pl.ds(s * ROWS_PER_TILE + i * CHUNK, CHUNK)])

        plsc.subcore_barrier()
        pltpu.sync_copy(dst_hbm.at[wid], dstv)

        @pl.loop(0, NCHUNK)
        def _scatter(j):
            pltpu.sync_copy(ones_v, acc.at[dstv.at[j]], add=True)

        plsc.subcore_barrier()
        pltpu.sync_copy(acc.at[pl.ds(s * ROWS_PER_TILE, ROWS_PER_TILE)],
                        out_hbm.at[c, pl.ds(s * ROWS_PER_TILE, ROWS_PER_TILE)])

    return deg_kernel(dst_r)


def _sc_hop(ytil, src_r, dst_r):
    @functools.partial(
        pl.kernel,
        out_type=jax.ShapeDtypeStruct((NC, NPAD, D), jnp.float32),
        mesh=_mesh(),
        scratch_types=[
            pltpu.VMEM((3, WIN, CHUNK), jnp.int32),
            pltpu.VMEM((NCHUNK, CHUNK), jnp.int32),
            pltpu.VMEM((2, CHUNK, D), jnp.float32),
            pltpu.VMEM_SHARED((NPAD, D), jnp.float32),
            pltpu.SemaphoreType.DMA,
            pltpu.SemaphoreType.DMA,
            pltpu.SemaphoreType.DMA,
            pltpu.SemaphoreType.DMA,
            pltpu.SemaphoreType.DMA,
            pltpu.SemaphoreType.DMA,
            pltpu.SemaphoreType.DMA,
            pltpu.SemaphoreType.DMA,
            pltpu.SemaphoreType.DMA,
        ],
    )
    def hop_kernel(ytil_hbm, src_hbm, dst_hbm, out_hbm, srcv, dstv, rows, acc,
                   g0, g1, g2, g3, s0, s1, i0, i1, i2):
        gsems = (g0, g1, g2, g3)
        ssems = (s0, s1)
        isems = (i0, i1, i2)
        c = lax.axis_index("c")
        s = lax.axis_index("s")
        wid = c * NS + s

        def fire_pair(p, slot, bank, t):
            for h in range(2):
                pltpu.async_copy(
                    ytil_hbm.at[srcv.at[bank, t, pl.ds(h * HCH, HCH)]],
                    rows.at[slot, pl.ds(h * HCH, HCH)], gsems[(2 * p + h) % 4])

        def wait_pair(p, slot, bank, t):
            for h in range(2):
                pltpu.make_async_copy(
                    ytil_hbm.at[srcv.at[bank, t, pl.ds(h * HCH, HCH)]],
                    rows.at[slot, pl.ds(h * HCH, HCH)],
                    gsems[(2 * p + h) % 4]).wait()

        @pl.loop(0, CHUNK)
        def _zfill(i):
            for k in range(D // 16):
                rows[0, i, pl.ds(k * 16, 16)] = jnp.zeros((16,), jnp.float32)

        @pl.loop(0, ROWS_PER_TILE // CHUNK)
        def _zero(i):
            pltpu.sync_copy(rows.at[0],
                            acc.at[pl.ds(s * ROWS_PER_TILE + i * CHUNK, CHUNK)])

        plsc.subcore_barrier()
        pltpu.sync_copy(dst_hbm.at[wid], dstv)
        pltpu.sync_copy(src_hbm.at[wid, pl.ds(0, WIN)], srcv.at[0])
        pltpu.async_copy(src_hbm.at[wid, pl.ds(WIN, WIN)], srcv.at[1], isems[1])
        fire_pair(0, 0, 0, 0)

        for w in range(NWINDOW):
            bank = w % 3
            if w + 2 < NWINDOW:
                pltpu.async_copy(src_hbm.at[wid, pl.ds((w + 2) * WIN, WIN)],
                                 srcv.at[(w + 2) % 3], isems[(w + 2) % 3])
            for t in range(WIN):
                p = w * WIN + t
                slot = p % 2
                wait_pair(p, slot, bank, t)
                pltpu.async_copy(rows.at[slot], acc.at[dstv.at[p]],
                                 ssems[slot], add=True)
                if p >= 1:
                    pltpu.make_async_copy(rows.at[1 - slot],
                                          acc.at[dstv.at[p - 1]],
                                          ssems[1 - slot]).wait()
                if p + 1 < NCHUNK:
                    if t + 1 < WIN:
                        fire_pair(p + 1, 1 - slot, bank, t + 1)
                    else:
                        nb = (w + 1) % 3
                        pltpu.make_async_copy(
                            src_hbm.at[wid, pl.ds((w + 1) * WIN, WIN)],
                            srcv.at[nb], isems[nb]).wait()
                        fire_pair(p + 1, 1 - slot, nb, 0)

        pltpu.make_async_copy(rows.at[(NCHUNK - 1) % 2],
                              acc.at[dstv.at[NCHUNK - 1]],
                              ssems[(NCHUNK - 1) % 2]).wait()
        plsc.subcore_barrier()
        pltpu.sync_copy(acc.at[pl.ds(s * ROWS_PER_TILE, ROWS_PER_TILE)],
                        out_hbm.at[c, pl.ds(s * ROWS_PER_TILE, ROWS_PER_TILE)])

    return hop_kernel(ytil, src_r, dst_r)


def _dinv_block(deg_ref, i):
    degs = deg_ref[0, :, 0:1] + deg_ref[1, :, 0:1] + 1.0
    rows = i * R + lax.broadcasted_iota(jnp.int32, (R, 1), 0)
    return jnp.where(rows < N_NODES, lax.rsqrt(degs), 0.0)


def _row_spec():
    return pl.BlockSpec((R, D), lambda i: (i, 0))


def _mat_spec():
    return pl.BlockSpec((D, D), lambda i: (0, 0))


def _bias_spec():
    return pl.BlockSpec((1, D), lambda i: (0, 0))


def _deg_spec():
    return pl.BlockSpec((NC, R, DEGW), lambda i: (0, i, 0))


def _acc_spec():
    return pl.BlockSpec((NC, R, D), lambda i: (0, i, 0))


def _tc_pre(xpad, W0, b0, W1, b1, deg):
    def body(x_ref, w0_ref, b0_ref, w1_ref, b1_ref, deg_ref, h0_ref, yt_ref):
        i = pl.program_id(0)
        dinv = _dinv_block(deg_ref, i)
        x = x_ref[...]
        h0 = jnp.maximum(
            jnp.dot(x, w0_ref[...], precision=lax.Precision.HIGHEST) + b0_ref[...],
            0.0)
        y1 = jnp.dot(h0, w1_ref[...], precision=lax.Precision.HIGHEST) + b1_ref[...]
        h0_ref[...] = h0
        yt_ref[...] = dinv * y1

    return pl.pallas_call(
        body,
        grid=(NPAD // R,),
        in_specs=[_row_spec(), _mat_spec(), _bias_spec(), _mat_spec(),
                  _bias_spec(), _deg_spec()],
        out_specs=[_row_spec(), _row_spec()],
        out_shape=[jax.ShapeDtypeStruct((NPAD, D), jnp.float32)] * 2,
    )(xpad, W0, b0.reshape(1, D), W1, b1.reshape(1, D), deg)


def _tc_mid(acc, yt, s_prev, deg, W, b):
    def body(acc_ref, yt_ref, s_ref, deg_ref, w_ref, b_ref, sout_ref, ytout_ref):
        i = pl.program_id(0)
        dinv = _dinv_block(deg_ref, i)
        a = acc_ref[0] + acc_ref[1] + yt_ref[...]
        x = jnp.maximum(dinv * a, 0.0)
        sout_ref[...] = s_ref[...] + x
        y = jnp.dot(x, w_ref[...], precision=lax.Precision.HIGHEST) + b_ref[...]
        ytout_ref[...] = dinv * y

    return pl.pallas_call(
        body,
        grid=(NPAD // R,),
        in_specs=[_acc_spec(), _row_spec(), _row_spec(), _deg_spec(),
                  _mat_spec(), _bias_spec()],
        out_specs=[_row_spec(), _row_spec()],
        out_shape=[jax.ShapeDtypeStruct((NPAD, D), jnp.float32)] * 2,
    )(acc, yt, s_prev, deg, W, b.reshape(1, D))


def _tc_final(acc, yt, s_prev, deg):
    def body(acc_ref, yt_ref, s_ref, deg_ref, out_ref):
        i = pl.program_id(0)
        dinv = _dinv_block(deg_ref, i)
        a = acc_ref[0] + acc_ref[1] + yt_ref[...]
        out_ref[...] = s_ref[...] + jnp.maximum(dinv * a, 0.0)

    return pl.pallas_call(
        body,
        grid=(NPAD // R,),
        in_specs=[_acc_spec(), _row_spec(), _row_spec(), _deg_spec()],
        out_specs=_row_spec(),
        out_shape=jax.ShapeDtypeStruct((NPAD, D), jnp.float32),
    )(acc, yt, s_prev, deg)


def kernel(features, edge_index, W0, b0, W1, b1, W2, b2, W3, b3):
    src = edge_index[0]
    dst = edge_index[1]
    fill = (N_NODES +
            jnp.arange(EPAD - N_EDGES, dtype=src.dtype) % (NPAD - N_NODES))
    src_pad = jnp.concatenate([src, fill])
    dst_pad = jnp.concatenate([dst, fill])
    src_r = src_pad.reshape(NW, NCHUNK, CHUNK)
    dst_r = dst_pad.reshape(NW, NCHUNK, CHUNK)
    dst_deg = dst_r
    xpad = jnp.zeros((NPAD, D), features.dtype).at[:N_NODES].set(features)

    deg = _sc_degree(dst_deg)
    s_run, yt = _tc_pre(xpad, W0, b0, W1, b1, deg)
    for (W, b) in ((W2, b2), (W3, b3)):
        acc = _sc_hop(yt, src_r, dst_r)
        s_run, yt = _tc_mid(acc, yt, s_run, deg, W, b)
    acc = _sc_hop(yt, src_r, dst_r)
    out = _tc_final(acc, yt, s_run, deg)
    return out[:N_NODES]


# speedup vs baseline: 23.8919x; 1.0204x over previous
import functools

import jax
import jax.numpy as jnp
from jax import lax
from jax.experimental import pallas as pl
from jax.experimental.pallas import tpu as pltpu
from jax.experimental.pallas import tpu_sc as plsc

N_NODES = 10000
N_EDGES = 320000
D = 128
NC, NS = 2, 16
NW = NC * NS
NPAD = 10240
CHUNK = 128
EPAD = 327680
NCHUNK = EPAD // NW // CHUNK
HCH = CHUNK // 2
ROWS_PER_TILE = NPAD // NS
DEGW = 16
WIN = 8
NWINDOW = NCHUNK // WIN
R = 1024


def _mesh():
    return plsc.VectorSubcoreMesh(core_axis_name="c", subcore_axis_name="s",
                                  num_cores=NC, num_subcores=NS)


def _sc_degree(dst_r):
    @functools.partial(
        pl.kernel,
        out_type=jax.ShapeDtypeStruct((NC, NPAD, DEGW), jnp.float32),
        mesh=_mesh(),
        scratch_types=[
            pltpu.VMEM((NCHUNK, CHUNK), jnp.int32),
            pltpu.VMEM((CHUNK, DEGW), jnp.float32),
            pltpu.VMEM((CHUNK, DEGW), jnp.float32),
            pltpu.VMEM_SHARED((NPAD, DEGW), jnp.float32),
        ],
    )
    def deg_kernel(dst_hbm, out_hbm, dstv, ones_v, zero_v, acc):
        c = lax.axis_index("c")
        s = lax.axis_index("s")
        wid = c * NS + s

        @pl.loop(0, CHUNK)
        def _fill(i):
            ones_v[i, :] = jnp.ones((DEGW,), jnp.float32)
            zero_v[i, :] = jnp.zeros((DEGW,), jnp.float32)

        @pl.loop(0, ROWS_PER_TILE // CHUNK)
        def _zero(i):
            pltpu.sync_copy(zero_v,
                            acc.at[pl.ds(s * ROWS_PER_TILE + i * CHUNK, CHUNK)])

        plsc.subcore_barrier()
        pltpu.sync_copy(dst_hbm.at[wid], dstv)

        @pl.loop(0, NCHUNK)
        def _scatter(j):
            pltpu.sync_copy(ones_v, acc.at[dstv.at[j]], add=True)

        plsc.subcore_barrier()
        pltpu.sync_copy(acc.at[pl.ds(s * ROWS_PER_TILE, ROWS_PER_TILE)],
                        out_hbm.at[c, pl.ds(s * ROWS_PER_TILE, ROWS_PER_TILE)])

    return deg_kernel(dst_r)


def _sc_hop(ytil, src_r, dst_r):
    @functools.partial(
        pl.kernel,
        out_type=jax.ShapeDtypeStruct((NC, NPAD, D), jnp.float32),
        mesh=_mesh(),
        scratch_types=[
            pltpu.VMEM((3, WIN, CHUNK), jnp.int32),
            pltpu.VMEM((NCHUNK, CHUNK), jnp.int32),
            pltpu.VMEM((2, CHUNK, D), jnp.float32),
            pltpu.VMEM_SHARED((NPAD, D), jnp.float32),
            pltpu.SemaphoreType.DMA,
            pltpu.SemaphoreType.DMA,
            pltpu.SemaphoreType.DMA,
            pltpu.SemaphoreType.DMA,
            pltpu.SemaphoreType.DMA,
            pltpu.SemaphoreType.DMA,
            pltpu.SemaphoreType.DMA,
            pltpu.SemaphoreType.DMA,
            pltpu.SemaphoreType.DMA,
        ],
    )
    def hop_kernel(ytil_hbm, src_hbm, dst_hbm, out_hbm, srcv, dstv, rows, acc,
                   g0, g1, g2, g3, s0, s1, i0, i1, i2):
        gsems = (g0, g1, g2, g3)
        ssems = (s0, s1)
        isems = (i0, i1, i2)
        c = lax.axis_index("c")
        s = lax.axis_index("s")
        wid = c * NS + s

        def fire_pair(p, slot, bank, t):
            for h in range(2):
                pltpu.async_copy(
                    ytil_hbm.at[srcv.at[bank, t, pl.ds(h * HCH, HCH)]],
                    rows.at[slot, pl.ds(h * HCH, HCH)], gsems[(2 * p + h) % 4])

        def wait_pair(p, slot, bank, t):
            for h in range(2):
                pltpu.make_async_copy(
                    ytil_hbm.at[srcv.at[bank, t, pl.ds(h * HCH, HCH)]],
                    rows.at[slot, pl.ds(h * HCH, HCH)],
                    gsems[(2 * p + h) % 4]).wait()

        @pl.loop(0, CHUNK)
        def _zfill(i):
            for k in range(D // 16):
                rows[0, i, pl.ds(k * 16, 16)] = jnp.zeros((16,), jnp.float32)

        @pl.loop(0, ROWS_PER_TILE // CHUNK)
        def _zero(i):
            pltpu.sync_copy(rows.at[0],
                            acc.at[pl.ds(s * ROWS_PER_TILE + i * CHUNK, CHUNK)])

        plsc.subcore_barrier()
        pltpu.sync_copy(dst_hbm.at[wid], dstv)
        pltpu.sync_copy(src_hbm.at[wid, pl.ds(0, WIN)], srcv.at[0])
        pltpu.async_copy(src_hbm.at[wid, pl.ds(WIN, WIN)], srcv.at[1], isems[1])
        fire_pair(0, 0, 0, 0)

        for w in range(NWINDOW):
            bank = w % 3
            if w + 2 < NWINDOW:
                pltpu.async_copy(src_hbm.at[wid, pl.ds((w + 2) * WIN, WIN)],
                                 srcv.at[(w + 2) % 3], isems[(w + 2) % 3])
            for t in range(WIN):
                p = w * WIN + t
                slot = p % 2
                wait_pair(p, slot, bank, t)
                pltpu.async_copy(rows.at[slot], acc.at[dstv.at[p]],
                                 ssems[slot], add=True)
                if p >= 1:
                    pltpu.make_async_copy(rows.at[1 - slot],
                                          acc.at[dstv.at[p - 1]],
                                          ssems[1 - slot]).wait()
                if p + 1 < NCHUNK:
                    if t + 1 < WIN:
                        fire_pair(p + 1, 1 - slot, bank, t + 1)
                    else:
                        nb = (w + 1) % 3
                        pltpu.make_async_copy(
                            src_hbm.at[wid, pl.ds((w + 1) * WIN, WIN)],
                            srcv.at[nb], isems[nb]).wait()
                        fire_pair(p + 1, 1 - slot, nb, 0)

        pltpu.make_async_copy(rows.at[(NCHUNK - 1) % 2],
                              acc.at[dstv.at[NCHUNK - 1]],
                              ssems[(NCHUNK - 1) % 2]).wait()
        plsc.subcore_barrier()
        pltpu.sync_copy(acc.at[pl.ds(s * ROWS_PER_TILE, ROWS_PER_TILE)],
                        out_hbm.at[c, pl.ds(s * ROWS_PER_TILE, ROWS_PER_TILE)])

    return hop_kernel(ytil, src_r, dst_r)


def _dinv_block(deg_ref, i):
    degs = deg_ref[0, :, 0:1] + deg_ref[1, :, 0:1] + 1.0
    rows = i * R + lax.broadcasted_iota(jnp.int32, (R, 1), 0)
    return jnp.where(rows < N_NODES, lax.rsqrt(degs), 0.0)


def _row_spec():
    return pl.BlockSpec((R, D), lambda i: (i, 0))


def _mat_spec():
    return pl.BlockSpec((D, D), lambda i: (0, 0))


def _bias_spec():
    return pl.BlockSpec((1, D), lambda i: (0, 0))


def _deg_spec():
    return pl.BlockSpec((NC, R, DEGW), lambda i: (0, i, 0))


def _acc_spec():
    return pl.BlockSpec((NC, R, D), lambda i: (0, i, 0))


def _tc_pre(xpad, W0, b0, W1, b1):
    def body(x_ref, w0_ref, b0_ref, w1_ref, b1_ref, h0_ref, y1_ref):
        x = x_ref[...]
        h0 = jnp.maximum(
            jnp.dot(x, w0_ref[...], precision=lax.Precision.HIGHEST) + b0_ref[...],
            0.0)
        y1_ref[...] = (
            jnp.dot(h0, w1_ref[...], precision=lax.Precision.HIGHEST) + b1_ref[...])
        h0_ref[...] = h0

    return pl.pallas_call(
        body,
        grid=(NPAD // R,),
        in_specs=[_row_spec(), _mat_spec(), _bias_spec(), _mat_spec(),
                  _bias_spec()],
        out_specs=[_row_spec(), _row_spec()],
        out_shape=[jax.ShapeDtypeStruct((NPAD, D), jnp.float32)] * 2,
    )(xpad, W0, b0.reshape(1, D), W1, b1.reshape(1, D))


def _tc_scale(y1, deg):
    def body(y_ref, deg_ref, yt_ref):
        i = pl.program_id(0)
        yt_ref[...] = _dinv_block(deg_ref, i) * y_ref[...]

    return pl.pallas_call(
        body,
        grid=(NPAD // R,),
        in_specs=[_row_spec(), _deg_spec()],
        out_specs=_row_spec(),
        out_shape=jax.ShapeDtypeStruct((NPAD, D), jnp.float32),
    )(y1, deg)


def _tc_mid(acc, yt, s_prev, deg, W, b):
    def body(acc_ref, yt_ref, s_ref, deg_ref, w_ref, b_ref, sout_ref, ytout_ref):
        i = pl.program_id(0)
        dinv = _dinv_block(deg_ref, i)
        a = acc_ref[0] + acc_ref[1] + yt_ref[...]
        x = jnp.maximum(dinv * a, 0.0)
        sout_ref[...] = s_ref[...] + x
        y = jnp.dot(x, w_ref[...], precision=lax.Precision.HIGHEST) + b_ref[...]
        ytout_ref[...] = dinv * y

    return pl.pallas_call(
        body,
        grid=(NPAD // R,),
        in_specs=[_acc_spec(), _row_spec(), _row_spec(), _deg_spec(),
                  _mat_spec(), _bias_spec()],
        out_specs=[_row_spec(), _row_spec()],
        out_shape=[jax.ShapeDtypeStruct((NPAD, D), jnp.float32)] * 2,
    )(acc, yt, s_prev, deg, W, b.reshape(1, D))


def _tc_final(acc, yt, s_prev, deg):
    def body(acc_ref, yt_ref, s_ref, deg_ref, out_ref):
        i = pl.program_id(0)
        dinv = _dinv_block(deg_ref, i)
        a = acc_ref[0] + acc_ref[1] + yt_ref[...]
        out_ref[...] = s_ref[...] + jnp.maximum(dinv * a, 0.0)

    return pl.pallas_call(
        body,
        grid=(NPAD // R,),
        in_specs=[_acc_spec(), _row_spec(), _row_spec(), _deg_spec()],
        out_specs=_row_spec(),
        out_shape=jax.ShapeDtypeStruct((NPAD, D), jnp.float32),
    )(acc, yt, s_prev, deg)


def kernel(features, edge_index, W0, b0, W1, b1, W2, b2, W3, b3):
    src = edge_index[0]
    dst = edge_index[1]
    fill = (N_NODES +
            jnp.arange(EPAD - N_EDGES, dtype=src.dtype) % (NPAD - N_NODES))
    src_pad = jnp.concatenate([src, fill])
    dst_pad = jnp.concatenate([dst, fill])
    src_r = src_pad.reshape(NW, NCHUNK, CHUNK)
    dst_r = dst_pad.reshape(NW, NCHUNK, CHUNK)
    dst_deg = dst_r
    xpad = jnp.zeros((NPAD, D), features.dtype).at[:N_NODES].set(features)

    deg = _sc_degree(dst_deg)
    s_run, y1 = _tc_pre(xpad, W0, b0, W1, b1)
    yt = _tc_scale(y1, deg)
    for (W, b) in ((W2, b2), (W3, b3)):
        acc = _sc_hop(yt, src_r, dst_r)
        s_run, yt = _tc_mid(acc, yt, s_run, deg, W, b)
    acc = _sc_hop(yt, src_r, dst_r)
    out = _tc_final(acc, yt, s_run, deg)
    return out[:N_NODES]
